# baseline (speedup 1.0000x reference)
import functools

import jax
import jax.numpy as jnp
from jax import lax
from jax.experimental import pallas as pl
from jax.experimental.pallas import tpu as pltpu
from jax.experimental.pallas import tpu_sc as plsc

_N = 1000
_B = 4096
_T = 20
_NC = 2
_NS = 16
_NW = _NC * _NS
_BPW = _B // _NW
_L = 16
_CSTART = (0, 248, 496, 744)
_CWIDTH = (248, 248, 248, 256)


@functools.partial(
    pl.kernel,
    out_type=jax.ShapeDtypeStruct((_T, _N, _B), jnp.float32),
    mesh=plsc.VectorSubcoreMesh(core_axis_name="c", subcore_axis_name="s"),
    compiler_params=pltpu.CompilerParams(needs_layout_passes=False),
    scratch_types=[
        pltpu.VMEM((_CWIDTH[0], _BPW), jnp.float32),
        pltpu.VMEM((_CWIDTH[1], _BPW), jnp.float32),
        pltpu.VMEM((_CWIDTH[2], _BPW), jnp.float32),
        pltpu.VMEM((_CWIDTH[3], _BPW), jnp.float32),
        pltpu.VMEM((_BPW * _T,), jnp.int32),
        pltpu.VMEM((_BPW,), jnp.int32),
        pltpu.VMEM((_BPW,), jnp.int32),
        pltpu.SemaphoreType.DMA,
        pltpu.SemaphoreType.DMA,
        pltpu.SemaphoreType.DMA,
        pltpu.SemaphoreType.DMA,
    ],
)
def _one_hot_sc(idx_hbm, zeros_hbm, out_hbm, buf0, buf1, buf2, buf3,
                idxall, cls0, cls1, sem0, sem1, sem2, sem3):
    wid = lax.axis_index("s") * _NC + lax.axis_index("c")
    b0 = wid * _BPW

    bufs = (buf0, buf1, buf2, buf3)
    sems = (sem0, sem1, sem2, sem3)
    clsv = (cls0, cls1)

    pltpu.sync_copy(idx_hbm.at[pl.ds(b0 * _T, _BPW * _T)], idxall)
    for c in range(4):
        pltpu.sync_copy(zeros_hbm.at[pl.ds(0, _CWIDTH[c]), :], bufs[c])

    lane = lax.broadcasted_iota(jnp.int32, (_L,), 0)
    ones16 = jnp.ones((_L,), jnp.float32)
    zeros16 = jnp.zeros((_L,), jnp.float32)

    def scatter_block(buf, clsref, c0, cw, val):
        for m in range(_BPW // _L):
            cls = clsref[pl.ds(m * _L, _L)]
            mask = (cls >= c0) & (cls < c0 + cw)
            plsc.store_scatter(buf, [cls - c0, lane + m * _L], val, mask=mask)

    copies = [None, None, None, None]
    for j in range(_T):
        cj = clsv[j % 2]
        for m in range(_BPW // _L):
            cj[pl.ds(m * _L, _L)] = plsc.load_gather(
                idxall, [(lane + m * _L) * _T + j])
        for c in range(4):
            buf, c0, cw = bufs[c], _CSTART[c], _CWIDTH[c]
            if copies[c] is not None:
                copies[c].wait()
                scatter_block(buf, clsv[(j - 1) % 2], c0, cw, zeros16)
            scatter_block(buf, cj, c0, cw, ones16)
            copies[c] = pltpu.async_copy(
                buf, out_hbm.at[j, pl.ds(c0, cw), pl.ds(b0, _BPW)], sems[c]
            )
    for c in range(4):
        copies[c].wait()


def kernel(indices, eye):
    del eye
    idx = indices.reshape(-1).astype(jnp.int32)
    zeros = jnp.zeros((_CWIDTH[3], _BPW), jnp.float32)
    out = _one_hot_sc(idx, zeros)
    return jnp.transpose(out, (2, 0, 1))


# speedup vs baseline: 1.0396x; 1.0396x over previous
import functools

import jax
import jax.numpy as jnp
from jax import lax
from jax.experimental import pallas as pl
from jax.experimental.pallas import tpu as pltpu
from jax.experimental.pallas import tpu_sc as plsc

_N = 1000
_B = 4096
_T = 20
_NC = 2
_NS = 16
_NW = _NC * _NS
_BPW = _B // _NW
_L = 16
_C0 = 488
_C1 = _N - _C0


@functools.partial(
    pl.kernel,
    out_type=jax.ShapeDtypeStruct((_T, _N, _B), jnp.float32),
    mesh=plsc.VectorSubcoreMesh(core_axis_name="c", subcore_axis_name="s"),
    compiler_params=pltpu.CompilerParams(needs_layout_passes=False),
    scratch_types=[
        pltpu.VMEM((_C0, _BPW), jnp.float32),
        pltpu.VMEM((_C1, _BPW), jnp.float32),
        pltpu.VMEM((_BPW * _T,), jnp.int32),
        pltpu.SemaphoreType.DMA,
        pltpu.SemaphoreType.DMA,
    ],
)
def _one_hot_sc(idx_hbm, zeros_hbm, out_hbm, bufa, bufb, idxall, sema, semb):
    wid = lax.axis_index("s") * _NC + lax.axis_index("c")
    b0 = wid * _BPW

    pltpu.sync_copy(idx_hbm.at[pl.ds(b0 * _T, _BPW * _T)], idxall)
    pltpu.sync_copy(zeros_hbm.at[pl.ds(0, _C0), :], bufa)
    pltpu.sync_copy(zeros_hbm.at[pl.ds(0, _C1), :], bufb)

    lane = lax.broadcasted_iota(jnp.int32, (_L,), 0)
    ones16 = jnp.ones((_L,), jnp.float32)
    zeros16 = jnp.zeros((_L,), jnp.float32)

    def scatter_block(buf, j, c0, cw, val):
        for m in range(_BPW // _L):
            bl = lane + m * _L
            cls = plsc.load_gather(idxall, [bl * _T + j])
            mask = (cls >= c0) & (cls < c0 + cw)
            plsc.store_scatter(buf, [cls - c0, bl], val, mask=mask)

    chunks = ((bufa, 0, _C0, sema), (bufb, _C0, _C1, semb))
    copies = [None, None]
    for j in range(_T):
        for s, (buf, c0, cw, sem) in enumerate(chunks):
            if copies[s] is not None:
                copies[s].wait()
                scatter_block(buf, j - 1, c0, cw, zeros16)
            scatter_block(buf, j, c0, cw, ones16)
            copies[s] = pltpu.async_copy(
                buf, out_hbm.at[j, pl.ds(c0, cw), pl.ds(b0, _BPW)], sem
            )
    copies[0].wait()
    copies[1].wait()


def kernel(indices, eye):
    del eye
    idx = indices.reshape(-1).astype(jnp.int32)
    zeros = jnp.zeros((_C1, _BPW), jnp.float32)
    out = _one_hot_sc(idx, zeros)
    return jnp.transpose(out, (2, 0, 1))
